# baseline (device time: 15767 ns/iter reference)
import jax
import jax.numpy as jnp
from jax import lax
from jax.experimental import pallas as pl
from jax.experimental.pallas import tpu as pltpu

N_DEV = 4


def _combine(bv, bi, cv, ci):
    take = (cv > bv) | ((cv == bv) & (ci < bi))
    return jnp.where(take, cv, bv), jnp.where(take, ci, bi)


def kernel(x):
    m_per, n = x.shape

    def body(x_ref, out_ref, comm_ref, send_sems, recv_sems):
        my_pos = lax.axis_index("i")
        p1 = my_pos ^ 1
        p2 = my_pos ^ 3

        barrier_sem = pltpu.get_barrier_semaphore()
        for nbr in [p1, p2]:
            pl.semaphore_signal(
                barrier_sem, inc=1,
                device_id=(nbr,), device_id_type=pl.DeviceIdType.MESH,
            )
        pl.semaphore_wait(barrier_sem, 2)

        xv = x_ref[:, :]
        val = jnp.max(xv, axis=0)
        row_iota = lax.broadcasted_iota(jnp.int32, (m_per, n), 0)
        idx_local = jnp.min(
            jnp.where(xv == val[None, :], row_iota, jnp.int32(2 * m_per * N_DEV)),
            axis=0,
        )
        idx = idx_local.astype(jnp.float32) + my_pos.astype(jnp.float32) * jnp.float32(
            m_per
        )

        comm_ref[0, 0, :] = val
        comm_ref[0, 1, :] = idx

        r1 = pltpu.make_async_remote_copy(
            src_ref=comm_ref.at[0],
            dst_ref=comm_ref.at[1],
            send_sem=send_sems.at[0],
            recv_sem=recv_sems.at[0],
            device_id=(p1,),
            device_id_type=pl.DeviceIdType.MESH,
        )
        r1.start()
        r1.wait()
        bv, bi = _combine(val, idx, comm_ref[1, 0, :], comm_ref[1, 1, :])

        comm_ref[0, 0, :] = bv
        comm_ref[0, 1, :] = bi
        r2 = pltpu.make_async_remote_copy(
            src_ref=comm_ref.at[0],
            dst_ref=comm_ref.at[2],
            send_sem=send_sems.at[1],
            recv_sem=recv_sems.at[1],
            device_id=(p2,),
            device_id_type=pl.DeviceIdType.MESH,
        )
        r2.start()
        r2.wait()
        bv, bi = _combine(bv, bi, comm_ref[2, 0, :], comm_ref[2, 1, :])

        out_ref[0, :] = bv
        out_ref[1, :] = bi

    return pl.pallas_call(
        body,
        out_shape=jax.ShapeDtypeStruct((2, n), jnp.float32),
        in_specs=[pl.BlockSpec(memory_space=pltpu.VMEM)],
        out_specs=pl.BlockSpec(memory_space=pltpu.VMEM),
        scratch_shapes=[
            pltpu.VMEM((3, 2, n), jnp.float32),
            pltpu.SemaphoreType.DMA((2,)),
            pltpu.SemaphoreType.DMA((2,)),
        ],
        compiler_params=pltpu.CompilerParams(collective_id=0),
    )(x)
